# Initial kernel scaffold; baseline (speedup 1.0000x reference)
import functools

import jax
import jax.numpy as jnp
from jax import lax
from jax.experimental import pallas as pl
from jax.experimental.pallas import tpu as pltpu
from jax.experimental.pallas import tpu_sc as plsc

_N = 10000
_E = 320000
_D = 128
_G = 64

_NC = 2
_NS = 16
_NW = _NC * _NS
_EPT = _E // _NW
_CH = 80
_NCHUNK = _EPT // _CH
_RPT = _N // _NS
_ZR = 125


def _agg_body(x_hbm, src_hbm, dst_hbm, out_hbm, src_v, dst_v, rows_v, buf_v,
              acc_sh, sem):
    c = lax.axis_index("c")
    s = lax.axis_index("s")
    wid = s * _NC + c

    def zstore(i, carry):
        buf_v[i // 8, pl.ds((i % 8) * 16, 16)] = jnp.zeros((16,), jnp.float32)
        return carry
    lax.fori_loop(0, _ZR * 8, zstore, 0)

    row0 = s * _RPT

    def zcopy(j, carry):
        pltpu.sync_copy(buf_v, acc_sh.at[pl.ds(row0 + j * _ZR, _ZR), :])
        return carry
    lax.fori_loop(0, _RPT // _ZR, zcopy, 0)
    plsc.subcore_barrier()

    base0 = wid * _EPT

    def edge_chunk(i, carry):
        base = base0 + i * _CH
        pltpu.sync_copy(src_hbm.at[pl.ds(base, _CH)], src_v)
        pltpu.sync_copy(dst_hbm.at[pl.ds(base, _CH)], dst_v)
        pltpu.async_copy(x_hbm.at[src_v], rows_v, sem).wait()
        pltpu.sync_copy(rows_v, acc_sh.at[dst_v], add=True)
        return carry
    lax.fori_loop(0, _NCHUNK, edge_chunk, 0)
    plsc.subcore_barrier()

    def wcopy(j, carry):
        r = row0 + j * _ZR
        pltpu.sync_copy(acc_sh.at[pl.ds(r, _ZR), :], buf_v)
        pltpu.sync_copy(buf_v, out_hbm.at[c, pl.ds(r, _ZR), :])
        return carry
    lax.fori_loop(0, _RPT // _ZR, wcopy, 0)


_agg = functools.partial(
    pl.kernel,
    out_type=jax.ShapeDtypeStruct((_NC, _N, _D), jnp.float32),
    mesh=plsc.VectorSubcoreMesh(core_axis_name="c", subcore_axis_name="s"),
    scratch_types=[
        pltpu.VMEM((_CH,), jnp.int32),
        pltpu.VMEM((_CH,), jnp.int32),
        pltpu.VMEM((_CH, _D), jnp.float32),
        pltpu.VMEM((_ZR, _D), jnp.float32),
        pltpu.VMEM_SHARED((_N, _D), jnp.float32),
        pltpu.SemaphoreType.DMA,
    ],
)(_agg_body)


_ROWS_BLK = 1000


def _layer_body(x_ref, p_ref, w1_ref, b1_ref, w2_ref, b2_ref, g_ref, bb_ref,
                o_ref):
    h = x_ref[...] + p_ref[0] + p_ref[1]
    h = jnp.maximum(
        jnp.dot(h, w1_ref[...], preferred_element_type=jnp.float32)
        + b1_ref[...], 0.0)
    h = jnp.dot(h, w2_ref[...], preferred_element_type=jnp.float32) + b2_ref[...]
    h = jnp.maximum(h, 0.0)
    mu = jnp.mean(h, axis=1, keepdims=True)
    var = jnp.mean((h - mu) * (h - mu), axis=1, keepdims=True)
    o_ref[...] = (h - mu) * lax.rsqrt(var + 1e-5) * g_ref[...] + bb_ref[...]


def _tc_layer(x, parts, W1, b1, W2, b2, g, bb):
    full = pl.BlockSpec((1, _D), lambda i: (0, 0))
    wspec = pl.BlockSpec((_D, _D), lambda i: (0, 0))
    return pl.pallas_call(
        _layer_body,
        grid=(_N // _ROWS_BLK,),
        in_specs=[
            pl.BlockSpec((_ROWS_BLK, _D), lambda i: (i, 0)),
            pl.BlockSpec((_NC, _ROWS_BLK, _D), lambda i: (0, i, 0)),
            wspec, full, wspec, full, full, full,
        ],
        out_specs=pl.BlockSpec((_ROWS_BLK, _D), lambda i: (i, 0)),
        out_shape=jax.ShapeDtypeStruct((_N, _D), jnp.float32),
    )(x, parts, W1, b1.reshape(1, _D), W2, b2.reshape(1, _D),
      g.reshape(1, _D), bb.reshape(1, _D))


def _final_body(x_ref, p_ref, w1_ref, b1_ref, w2_ref, b2_ref, batch_ref,
                p1w_ref, p1b_ref, p2w_ref, p2b_ref, o_ref):
    h = x_ref[...] + p_ref[0] + p_ref[1]
    h = jnp.maximum(
        jnp.dot(h, w1_ref[...], preferred_element_type=jnp.float32)
        + b1_ref[...], 0.0)
    h = jnp.dot(h, w2_ref[...], preferred_element_type=jnp.float32) + b2_ref[...]
    h = jnp.maximum(h, 0.0)
    gid = lax.broadcasted_iota(jnp.int32, (_N, _G), 1)
    onehot = (batch_ref[...] == gid).astype(jnp.float32)
    sums = lax.dot_general(onehot, h, (((0,), (0,)), ((), ())),
                           preferred_element_type=jnp.float32)
    cnt = lax.dot_general(onehot, jnp.ones((_N, 1), jnp.float32),
                          (((0,), (0,)), ((), ())),
                          preferred_element_type=jnp.float32)
    pooled = sums / jnp.maximum(cnt, 1.0)
    out = jnp.dot(pooled, p1w_ref[...],
                  preferred_element_type=jnp.float32) + p1b_ref[...]
    o_ref[...] = jnp.dot(out, p2w_ref[...],
                         preferred_element_type=jnp.float32) + p2b_ref[...]


def _tc_final(x, parts, W1, b1, W2, b2, batch2d, p1_W, p1_b, p2_W, p2_b):
    return pl.pallas_call(
        _final_body,
        out_shape=jax.ShapeDtypeStruct((_G, 1), jnp.float32),
    )(x, parts, W1, b1.reshape(1, _D), W2, b2.reshape(1, _D), batch2d,
      p1_W, p1_b.reshape(1, _D), p2_W, p2_b.reshape(1, 1))


def kernel(x, edge_index, batch, c0_W1, c0_b1, c0_W2, c0_b2, c1_W1, c1_b1,
           c1_W2, c1_b2, c2_W1, c2_b1, c2_W2, c2_b2, ln0_g, ln0_b, ln1_g,
           ln1_b, p1_W, p1_b, p2_W, p2_b):
    src = edge_index[0]
    dst = edge_index[1]
    batch2d = batch.reshape(_N, 1)

    p0 = _agg(x, src, dst)
    x1 = _tc_layer(x, p0, c0_W1, c0_b1, c0_W2, c0_b2, ln0_g, ln0_b)
    p1 = _agg(x1, src, dst)
    x2 = _tc_layer(x1, p1, c1_W1, c1_b1, c1_W2, c1_b2, ln1_g, ln1_b)
    p2 = _agg(x2, src, dst)
    return _tc_final(x2, p2, c2_W1, c2_b1, c2_W2, c2_b2, batch2d,
                     p1_W, p1_b, p2_W, p2_b)


# speedup vs baseline: 4.6692x; 4.6692x over previous
import functools

import jax
import jax.numpy as jnp
from jax import lax
from jax.experimental import pallas as pl
from jax.experimental.pallas import tpu as pltpu
from jax.experimental.pallas import tpu_sc as plsc

_N = 10000
_E = 320000
_D = 128
_G = 64

_NC = 2
_NS = 16
_NW = _NC * _NS
_EPT = _E // _NW
_CH = 80
_NCHUNK = _EPT // _CH
_NPAD = 10240
_RPT = _NPAD // _NS
_ZR = 128


def _agg_body(x_hbm, src_hbm, dst_hbm, out_hbm, src_v, dst_v, rows_v, buf_v,
              acc_sh, sem):
    c = lax.axis_index("c")
    s = lax.axis_index("s")
    wid = s * _NC + c

    def zstore(i, carry):
        buf_v[i // 8, pl.ds((i % 8) * 16, 16)] = jnp.zeros((16,), jnp.float32)
        return carry
    lax.fori_loop(0, _ZR * 8, zstore, 0)

    row0 = s * _RPT

    def zcopy(j, carry):
        pltpu.sync_copy(buf_v, acc_sh.at[pl.ds(row0 + j * _ZR, _ZR), :])
        return carry
    lax.fori_loop(0, _RPT // _ZR, zcopy, 0)
    plsc.subcore_barrier()

    base0 = wid * _EPT

    def edge_chunk(i, carry):
        base = base0 + i * _CH
        pltpu.sync_copy(src_hbm.at[pl.ds(base, _CH)], src_v)
        pltpu.sync_copy(dst_hbm.at[pl.ds(base, _CH)], dst_v)
        pltpu.async_copy(x_hbm.at[src_v], rows_v, sem).wait()
        pltpu.sync_copy(rows_v, acc_sh.at[dst_v], add=True)
        return carry
    lax.fori_loop(0, _NCHUNK, edge_chunk, 0)
    plsc.subcore_barrier()

    def wcopy(j, carry):
        r = row0 + j * _ZR
        pltpu.sync_copy(acc_sh.at[pl.ds(r, _ZR), :], buf_v)
        pltpu.sync_copy(buf_v, out_hbm.at[c, pl.ds(r, _ZR), :])
        return carry
    lax.fori_loop(0, _RPT // _ZR, wcopy, 0)


_agg = functools.partial(
    pl.kernel,
    out_type=jax.ShapeDtypeStruct((_NC, _NPAD, _D), jnp.float32),
    mesh=plsc.VectorSubcoreMesh(core_axis_name="c", subcore_axis_name="s"),
    scratch_types=[
        pltpu.VMEM((_CH,), jnp.int32),
        pltpu.VMEM((_CH,), jnp.int32),
        pltpu.VMEM((_CH, _D), jnp.float32),
        pltpu.VMEM((_ZR, _D), jnp.float32),
        pltpu.VMEM_SHARED((_NPAD, _D), jnp.float32),
        pltpu.SemaphoreType.DMA,
    ],
)(_agg_body)


_ROWS_BLK = 1000


def _layer_body(x_ref, p_ref, w1_ref, b1_ref, w2_ref, b2_ref, g_ref, bb_ref,
                o_ref):
    h = x_ref[...] + p_ref[0] + p_ref[1]
    h = jnp.maximum(
        jnp.dot(h, w1_ref[...], preferred_element_type=jnp.float32)
        + b1_ref[...], 0.0)
    h = jnp.dot(h, w2_ref[...], preferred_element_type=jnp.float32) + b2_ref[...]
    h = jnp.maximum(h, 0.0)
    mu = jnp.mean(h, axis=1, keepdims=True)
    var = jnp.mean((h - mu) * (h - mu), axis=1, keepdims=True)
    o_ref[...] = (h - mu) * lax.rsqrt(var + 1e-5) * g_ref[...] + bb_ref[...]


def _tc_layer(x, parts, W1, b1, W2, b2, g, bb):
    full = pl.BlockSpec((1, _D), lambda i: (0, 0))
    wspec = pl.BlockSpec((_D, _D), lambda i: (0, 0))
    return pl.pallas_call(
        _layer_body,
        grid=(_N // _ROWS_BLK,),
        in_specs=[
            pl.BlockSpec((_ROWS_BLK, _D), lambda i: (i, 0)),
            pl.BlockSpec((_NC, _ROWS_BLK, _D), lambda i: (0, i, 0)),
            wspec, full, wspec, full, full, full,
        ],
        out_specs=pl.BlockSpec((_ROWS_BLK, _D), lambda i: (i, 0)),
        out_shape=jax.ShapeDtypeStruct((_N, _D), jnp.float32),
    )(x, parts, W1, b1.reshape(1, _D), W2, b2.reshape(1, _D),
      g.reshape(1, _D), bb.reshape(1, _D))


def _final_body(x_ref, p_ref, w1_ref, b1_ref, w2_ref, b2_ref, batch_ref,
                p1w_ref, p1b_ref, p2w_ref, p2b_ref, o_ref):
    h = x_ref[...] + p_ref[0, :_N, :] + p_ref[1, :_N, :]
    h = jnp.maximum(
        jnp.dot(h, w1_ref[...], preferred_element_type=jnp.float32)
        + b1_ref[...], 0.0)
    h = jnp.dot(h, w2_ref[...], preferred_element_type=jnp.float32) + b2_ref[...]
    h = jnp.maximum(h, 0.0)
    gid = lax.broadcasted_iota(jnp.int32, (_N, _G), 1)
    onehot = (batch_ref[...] == gid).astype(jnp.float32)
    sums = lax.dot_general(onehot, h, (((0,), (0,)), ((), ())),
                           preferred_element_type=jnp.float32)
    cnt = lax.dot_general(onehot, jnp.ones((_N, 1), jnp.float32),
                          (((0,), (0,)), ((), ())),
                          preferred_element_type=jnp.float32)
    pooled = sums / jnp.maximum(cnt, 1.0)
    out = jnp.dot(pooled, p1w_ref[...],
                  preferred_element_type=jnp.float32) + p1b_ref[...]
    o_ref[...] = jnp.dot(out, p2w_ref[...],
                         preferred_element_type=jnp.float32) + p2b_ref[...]


def _tc_final(x, parts, W1, b1, W2, b2, batch2d, p1_W, p1_b, p2_W, p2_b):
    return pl.pallas_call(
        _final_body,
        out_shape=jax.ShapeDtypeStruct((_G, 1), jnp.float32),
    )(x, parts, W1, b1.reshape(1, _D), W2, b2.reshape(1, _D), batch2d,
      p1_W, p1_b.reshape(1, _D), p2_W, p2_b.reshape(1, 1))


def kernel(x, edge_index, batch, c0_W1, c0_b1, c0_W2, c0_b2, c1_W1, c1_b1,
           c1_W2, c1_b2, c2_W1, c2_b1, c2_W2, c2_b2, ln0_g, ln0_b, ln1_g,
           ln1_b, p1_W, p1_b, p2_W, p2_b):
    src = edge_index[0]
    dst = edge_index[1]
    batch2d = batch.reshape(_N, 1)

    p0 = _agg(x, src, dst)
    x1 = _tc_layer(x, p0, c0_W1, c0_b1, c0_W2, c0_b2, ln0_g, ln0_b)
    p1 = _agg(x1, src, dst)
    x2 = _tc_layer(x1, p1, c1_W1, c1_b1, c1_W2, c1_b2, ln1_g, ln1_b)
    p2 = _agg(x2, src, dst)
    return _tc_final(x2, p2, c2_W1, c2_b1, c2_W2, c2_b2, batch2d,
                     p1_W, p1_b, p2_W, p2_b)


# speedup vs baseline: 10.1626x; 2.1765x over previous
import functools

import jax
import jax.numpy as jnp
from jax import lax
from jax.experimental import pallas as pl
from jax.experimental.pallas import tpu as pltpu
from jax.experimental.pallas import tpu_sc as plsc

_N = 10000
_E = 320000
_D = 128
_G = 64

_NC = 2
_NS = 16
_NW = _NC * _NS
_EPT = _E // _NW
_CH = 80
_NCHUNK = _EPT // _CH
_KB = 25
_NGRP = _NCHUNK // _KB
_NBUF = 3
_LA = 2
_NPAD = 10240
_RPT = _NPAD // _NS
_ZR = 80


def _agg_body(x_hbm, src_hbm, dst_hbm, out_hbm, srcs_v, dsts_v,
              rows0, rows1, rows2, acc_sh, g0, g1, g2):
    c = lax.axis_index("c")
    s = lax.axis_index("s")
    wid = s * _NC + c
    rows = [rows0, rows1, rows2]
    gsem = [g0, g1, g2]

    def zstore(i, carry):
        rows0[i // 8, pl.ds((i % 8) * 16, 16)] = jnp.zeros((16,), jnp.float32)
        return carry
    lax.fori_loop(0, _ZR * 8, zstore, 0)

    row0 = s * _RPT

    def zcopy(j, carry):
        pltpu.sync_copy(rows0, acc_sh.at[pl.ds(row0 + j * _ZR, _ZR), :])
        return carry
    lax.fori_loop(0, _RPT // _ZR, zcopy, 0)
    plsc.subcore_barrier()

    def wait_gather(u, b):
        pltpu.make_async_copy(x_hbm.at[srcs_v.at[u]], rows[b],
                              gsem[b]).wait()

    def issue_gather(u, b):
        pltpu.async_copy(x_hbm.at[srcs_v.at[u]], rows[b], gsem[b])

    def group(g, carry):
        pltpu.sync_copy(src_hbm.at[wid, g], srcs_v)
        pltpu.sync_copy(dst_hbm.at[wid, g], dsts_v)
        for u in range(_LA):
            issue_gather(u, u)
        for u in range(_KB):
            b = u % _NBUF
            wait_gather(u, b)
            pltpu.sync_copy(rows[b], acc_sh.at[dsts_v.at[u]], add=True)
            if u + _LA < _KB:
                issue_gather(u + _LA, (u + _LA) % _NBUF)
        return carry
    lax.fori_loop(0, _NGRP, group, 0)
    plsc.subcore_barrier()

    def wcopy(j, carry):
        r = row0 + j * _ZR
        pltpu.sync_copy(acc_sh.at[pl.ds(r, _ZR), :], rows0)
        pltpu.sync_copy(rows0, out_hbm.at[c, pl.ds(r, _ZR), :])
        return carry
    lax.fori_loop(0, _RPT // _ZR, wcopy, 0)


_agg = functools.partial(
    pl.kernel,
    out_type=jax.ShapeDtypeStruct((_NC, _NPAD, _D), jnp.float32),
    mesh=plsc.VectorSubcoreMesh(core_axis_name="c", subcore_axis_name="s"),
    scratch_types=[
        pltpu.VMEM((_KB, _CH), jnp.int32),
        pltpu.VMEM((_KB, _CH), jnp.int32),
        pltpu.VMEM((_CH, _D), jnp.float32),
        pltpu.VMEM((_CH, _D), jnp.float32),
        pltpu.VMEM((_CH, _D), jnp.float32),
        pltpu.VMEM_SHARED((_NPAD, _D), jnp.float32),
        pltpu.SemaphoreType.DMA,
        pltpu.SemaphoreType.DMA,
        pltpu.SemaphoreType.DMA,
    ],
)(_agg_body)


_ROWS_BLK = 1000


def _layer_body(x_ref, p_ref, w1_ref, b1_ref, w2_ref, b2_ref, g_ref, bb_ref,
                o_ref):
    h = x_ref[...] + p_ref[0] + p_ref[1]
    h = jnp.maximum(
        jnp.dot(h, w1_ref[...], preferred_element_type=jnp.float32)
        + b1_ref[...], 0.0)
    h = jnp.dot(h, w2_ref[...], preferred_element_type=jnp.float32) + b2_ref[...]
    h = jnp.maximum(h, 0.0)
    mu = jnp.mean(h, axis=1, keepdims=True)
    var = jnp.mean((h - mu) * (h - mu), axis=1, keepdims=True)
    o_ref[...] = (h - mu) * lax.rsqrt(var + 1e-5) * g_ref[...] + bb_ref[...]


def _tc_layer(x, parts, W1, b1, W2, b2, g, bb):
    full = pl.BlockSpec((1, _D), lambda i: (0, 0))
    wspec = pl.BlockSpec((_D, _D), lambda i: (0, 0))
    return pl.pallas_call(
        _layer_body,
        grid=(_N // _ROWS_BLK,),
        in_specs=[
            pl.BlockSpec((_ROWS_BLK, _D), lambda i: (i, 0)),
            pl.BlockSpec((_NC, _ROWS_BLK, _D), lambda i: (0, i, 0)),
            wspec, full, wspec, full, full, full,
        ],
        out_specs=pl.BlockSpec((_ROWS_BLK, _D), lambda i: (i, 0)),
        out_shape=jax.ShapeDtypeStruct((_N, _D), jnp.float32),
    )(x, parts, W1, b1.reshape(1, _D), W2, b2.reshape(1, _D),
      g.reshape(1, _D), bb.reshape(1, _D))


def _final_body(x_ref, p_ref, w1_ref, b1_ref, w2_ref, b2_ref, batch_ref,
                p1w_ref, p1b_ref, p2w_ref, p2b_ref, o_ref):
    h = x_ref[...] + p_ref[0, :_N, :] + p_ref[1, :_N, :]
    h = jnp.maximum(
        jnp.dot(h, w1_ref[...], preferred_element_type=jnp.float32)
        + b1_ref[...], 0.0)
    h = jnp.dot(h, w2_ref[...], preferred_element_type=jnp.float32) + b2_ref[...]
    h = jnp.maximum(h, 0.0)
    gid = lax.broadcasted_iota(jnp.int32, (_N, _G), 1)
    onehot = (batch_ref[...] == gid).astype(jnp.float32)
    sums = lax.dot_general(onehot, h, (((0,), (0,)), ((), ())),
                           preferred_element_type=jnp.float32)
    cnt = lax.dot_general(onehot, jnp.ones((_N, 1), jnp.float32),
                          (((0,), (0,)), ((), ())),
                          preferred_element_type=jnp.float32)
    pooled = sums / jnp.maximum(cnt, 1.0)
    out = jnp.dot(pooled, p1w_ref[...],
                  preferred_element_type=jnp.float32) + p1b_ref[...]
    o_ref[...] = jnp.dot(out, p2w_ref[...],
                         preferred_element_type=jnp.float32) + p2b_ref[...]


def _tc_final(x, parts, W1, b1, W2, b2, batch2d, p1_W, p1_b, p2_W, p2_b):
    return pl.pallas_call(
        _final_body,
        out_shape=jax.ShapeDtypeStruct((_G, 1), jnp.float32),
    )(x, parts, W1, b1.reshape(1, _D), W2, b2.reshape(1, _D), batch2d,
      p1_W, p1_b.reshape(1, _D), p2_W, p2_b.reshape(1, 1))


def kernel(x, edge_index, batch, c0_W1, c0_b1, c0_W2, c0_b2, c1_W1, c1_b1,
           c1_W2, c1_b2, c2_W1, c2_b1, c2_W2, c2_b2, ln0_g, ln0_b, ln1_g,
           ln1_b, p1_W, p1_b, p2_W, p2_b):
    src = edge_index[0].reshape(_NW, _NGRP, _KB, _CH)
    dst = edge_index[1].reshape(_NW, _NGRP, _KB, _CH)
    batch2d = batch.reshape(_N, 1)

    p0 = _agg(x, src, dst)
    x1 = _tc_layer(x, p0, c0_W1, c0_b1, c0_W2, c0_b2, ln0_g, ln0_b)
    p1 = _agg(x1, src, dst)
    x2 = _tc_layer(x1, p1, c1_W1, c1_b1, c1_W2, c1_b2, ln1_g, ln1_b)
    p2 = _agg(x2, src, dst)
    return _tc_final(x2, p2, c2_W1, c2_b1, c2_W2, c2_b2, batch2d,
                     p1_W, p1_b, p2_W, p2_b)
